# baseline (speedup 1.0000x reference)
import functools

import jax
import jax.numpy as jnp
from jax.experimental import pallas as pl
from jax.experimental.pallas import tpu as pltpu
from jax.experimental.pallas import tpu_sc as plsc

H = 128
W = 128
N = 2048
HB = 16
NB = H // HB
NSEG = 4
SEG = N // NSEG
PR = 16
PC = N // PR
UNROLL = 6
SPAD = 8
CAP = SEG + SPAD

_INV255 = 1.0 / 255.0


def _prep_body(mx_ref, my_ref, op_ref, sx_ref, sy_ref, th_ref,
               a2_ref, b2_ref, c2_ref, b0_ref, b1_ref, radii_ref):
    th = th_ref[...]
    c = jnp.cos(th)
    s = jnp.sin(th)
    sx2 = sx_ref[...] ** 2
    sy2 = sy_ref[...] ** 2
    Sxx = c * c * sx2 + s * s * sy2 + 0.3
    Sxy = c * s * (sx2 - sy2)
    Syy = s * s * sx2 + c * c * sy2 + 0.3
    det = Sxx * Syy - Sxy * Sxy
    inv_det = 1.0 / det
    a2_ref[...] = -0.5 * Syy * inv_det
    b2_ref[...] = Sxy * inv_det
    c2_ref[...] = -0.5 * Sxx * inv_det
    mid = 0.5 * (Sxx + Syy)
    lam = mid + jnp.sqrt(jnp.maximum(mid * mid - det, 0.1))
    radii_ref[...] = jnp.ceil(3.0 * jnp.sqrt(lam)).astype(jnp.int32)
    op = op_ref[...]
    log_t = jnp.log(jnp.maximum(op, 1e-30) * 255.0)
    rcut = jnp.sqrt(2.0 * Syy * jnp.maximum(log_t, 0.0)) * 1.001 + 0.01
    my = my_ref[...]
    ylo = jnp.maximum(jnp.ceil(my - 0.5 - rcut), 0.0)
    yhi = jnp.minimum(jnp.floor(my - 0.5 + rcut), float(H - 1))
    empty = ylo > yhi
    b0 = (ylo.astype(jnp.int32) // HB)
    b1 = (yhi.astype(jnp.int32) // HB)
    b0_ref[...] = jnp.where(empty, NB + 1, b0)
    b1_ref[...] = jnp.where(empty, 0, b1)
    _ = mx_ref


def _bin_body(b0_hbm, b1_hbm, idx_hbm, cnt_hbm, b0_v, b1_v, idx_v,
              cnt_v, sem):
    c = jax.lax.axis_index("c")
    s = jax.lax.axis_index("s")
    u = s * 2 + c
    band_local = u // NSEG
    band = band_local
    seg = u % NSEG
    gbase = seg * SEG
    pltpu.sync_copy(b0_hbm.at[pl.ds(gbase, SEG)], b0_v)
    pltpu.sync_copy(b1_hbm.at[pl.ds(gbase, SEG)], b1_v)

    def chunk(i, ptr):
        b0c = b0_v[pl.ds(i * 16, 16)]
        b1c = b1_v[pl.ds(i * 16, 16)]
        mask = (b0c <= band) & (band <= b1c)
        mi = jnp.where(mask, 1, 0).astype(jnp.int32)
        pos = jax.lax.cumsum(mi, axis=0)
        offs = pos + (ptr - 1)
        gidx = jax.lax.iota(jnp.int32, 16) + (gbase + i * 16)
        plsc.store_scatter(idx_v, [offs], gidx, mask=mask)
        return ptr + jnp.sum(mi)

    ptr = jax.lax.fori_loop(0, SEG // 16, chunk, jnp.int32(0))
    lane = jax.lax.iota(jnp.int32, 16)
    plsc.store_scatter(idx_v, [ptr + lane], jnp.full((16,), N, jnp.int32),
                       mask=lane < SPAD)
    cnt_v[...] = jnp.full((16,), ptr, jnp.int32)
    pltpu.sync_copy(idx_v, idx_hbm.at[band_local, seg])
    pltpu.sync_copy(cnt_v, cnt_hbm.at[band_local, seg])
    _ = sem


@functools.lru_cache(maxsize=1)
def _make_bin_lists():
    return pl.kernel(
        _bin_body,
        out_type=(
            jax.ShapeDtypeStruct((NB, NSEG, CAP), jnp.int32),
            jax.ShapeDtypeStruct((NB, NSEG, 16), jnp.int32),
        ),
        mesh=plsc.VectorSubcoreMesh(core_axis_name="c", subcore_axis_name="s"),
        compiler_params=pltpu.CompilerParams(needs_layout_passes=False),
        scratch_types=[
            pltpu.VMEM((SEG,), jnp.int32),
            pltpu.VMEM((SEG,), jnp.int32),
            pltpu.VMEM((CAP,), jnp.int32),
            pltpu.VMEM((16,), jnp.int32),
            pltpu.SemaphoreType.DMA,
        ],
    )


def _bin_lists(b0, b1):
    return _make_bin_lists()(b0, b1)


def _raster_body(a2_ref, b2_ref, c2_ref, mx_ref, my_ref, op_ref,
                 cr_ref, cg_ref, cb_ref, idx_ref, cnt_ref, bg_ref, out_ref):
    b = pl.program_id(0)
    y0 = (b * HB).astype(jnp.float32) + 0.5
    py = jax.lax.broadcasted_iota(jnp.int32, (HB, W), 0).astype(jnp.float32) + y0
    px = jax.lax.broadcasted_iota(jnp.int32, (HB, W), 1).astype(jnp.float32) + 0.5

    def group(seg, jg, carry):
        T, ra, ga, ba = carry
        als = []
        cols = []
        base = jg * UNROLL
        for k in range(UNROLL):
            g = idx_ref[b, seg, base + k]
            dx = px - mx_ref[g]
            dy = py - my_ref[g]
            pw = dx * dx * a2_ref[g] + dy * dy * c2_ref[g] + dx * dy * b2_ref[g]
            al = op_ref[g] * jnp.exp(pw)
            al = jnp.where(al < _INV255, 0.0, al)
            als.append(al)
            cols.append((cr_ref[g], cg_ref[g], cb_ref[g]))
        q = [1.0 - al for al in als]
        cache = {}

        def rp(i, j):
            if (i, j) not in cache:
                if j - i == 1:
                    cache[(i, j)] = q[i]
                else:
                    m = i + (1 << ((j - i - 1).bit_length() - 1))
                    cache[(i, j)] = rp(i, m) * rp(m, j)
            return cache[(i, j)]

        us = [als[0]] + [als[k] * rp(0, k) for k in range(1, UNROLL)]

        def tree_sum(vals):
            vals = list(vals)
            while len(vals) > 1:
                nxt = [vals[i] + vals[i + 1] for i in range(0, len(vals) - 1, 2)]
                if len(vals) % 2:
                    nxt.append(vals[-1])
                vals = nxt
            return vals[0]

        sr = tree_sum(us[k] * cols[k][0] for k in range(UNROLL))
        sg = tree_sum(us[k] * cols[k][1] for k in range(UNROLL))
        sb = tree_sum(us[k] * cols[k][2] for k in range(UNROLL))
        ra = ra + T * sr
        ga = ga + T * sg
        ba = ba + T * sb
        T = T * rp(0, UNROLL)
        return (T, ra, ga, ba)

    ones = jnp.ones((HB, W), jnp.float32)
    zeros = jnp.zeros((HB, W), jnp.float32)
    carry = (ones, zeros, zeros, zeros)
    for seg in range(NSEG):
        n = cnt_ref[b, seg, 0]
        ngroups = (n + UNROLL - 1) // UNROLL
        carry = jax.lax.fori_loop(0, ngroups, functools.partial(group, seg),
                                  carry)
    T, ra, ga, ba = carry
    out_ref[0] = ra + T * bg_ref[0]
    out_ref[1] = ga + T * bg_ref[1]
    out_ref[2] = ba + T * bg_ref[2]


def kernel(means2D, opacities, colors, scale, rots, bg):
    f32 = jnp.float32
    mx2 = means2D[:, 0].reshape(PR, PC)
    my2 = means2D[:, 1].reshape(PR, PC)
    op2 = opacities[:, 0].reshape(PR, PC)
    sx2 = scale[:, 0].reshape(PR, PC)
    sy2 = scale[:, 1].reshape(PR, PC)
    th2 = rots[:, 0].reshape(PR, PC)

    a2, b2, c2, b0, b1, radii2 = pl.pallas_call(
        _prep_body,
        out_shape=(
            jax.ShapeDtypeStruct((PR, PC), f32),
            jax.ShapeDtypeStruct((PR, PC), f32),
            jax.ShapeDtypeStruct((PR, PC), f32),
            jax.ShapeDtypeStruct((PR, PC), jnp.int32),
            jax.ShapeDtypeStruct((PR, PC), jnp.int32),
            jax.ShapeDtypeStruct((PR, PC), jnp.int32),
        ),
    )(mx2, my2, op2, sx2, sy2, th2)

    b0r = b0.reshape(N)
    b1r = b1.reshape(N)

    pad = jnp.zeros((8,), f32)
    def _p(x):
        return jnp.concatenate([x.reshape(N), pad])
    params = (
        _p(a2), _p(b2), _p(c2),
        _p(mx2), _p(my2), _p(op2),
        _p(colors[:, 0]), _p(colors[:, 1]), _p(colors[:, 2]),
    )

    smem = pl.BlockSpec(memory_space=pltpu.SMEM)

    idx, cnt = _bin_lists(b0r, b1r)
    out = pl.pallas_call(
        _raster_body,
        grid=(NB,),
        in_specs=[smem] * 12,
        out_specs=pl.BlockSpec((3, HB, W), lambda b: (0, b, 0)),
        out_shape=jax.ShapeDtypeStruct((3, H, W), f32),
        compiler_params=pltpu.CompilerParams(
            dimension_semantics=("arbitrary",),
        ),
    )(*params, idx, cnt, bg)
    return (out, radii2.reshape(N))


# speedup vs baseline: 1.0482x; 1.0482x over previous
import functools

import jax
import jax.numpy as jnp
from jax.experimental import pallas as pl
from jax.experimental.pallas import tpu as pltpu
from jax.experimental.pallas import tpu_sc as plsc

H = 128
W = 128
N = 2048
HB = 16
NB = H // HB
NSEG = 4
SEG = N // NSEG
PR = 16
PC = N // PR
UNROLL = 12
SPAD = 16
CAP = SEG + SPAD

_INV255 = 1.0 / 255.0


def _prep_body(mx_ref, my_ref, op_ref, sx_ref, sy_ref, th_ref,
               a2_ref, b2_ref, c2_ref, b0_ref, b1_ref, radii_ref):
    th = th_ref[...]
    c = jnp.cos(th)
    s = jnp.sin(th)
    sx2 = sx_ref[...] ** 2
    sy2 = sy_ref[...] ** 2
    Sxx = c * c * sx2 + s * s * sy2 + 0.3
    Sxy = c * s * (sx2 - sy2)
    Syy = s * s * sx2 + c * c * sy2 + 0.3
    det = Sxx * Syy - Sxy * Sxy
    inv_det = 1.0 / det
    a2_ref[...] = -0.5 * Syy * inv_det
    b2_ref[...] = Sxy * inv_det
    c2_ref[...] = -0.5 * Sxx * inv_det
    mid = 0.5 * (Sxx + Syy)
    lam = mid + jnp.sqrt(jnp.maximum(mid * mid - det, 0.1))
    radii_ref[...] = jnp.ceil(3.0 * jnp.sqrt(lam)).astype(jnp.int32)
    op = op_ref[...]
    log_t = jnp.log(jnp.maximum(op, 1e-30) * 255.0)
    rcut = jnp.sqrt(2.0 * Syy * jnp.maximum(log_t, 0.0)) * 1.001 + 0.01
    my = my_ref[...]
    ylo = jnp.maximum(jnp.ceil(my - 0.5 - rcut), 0.0)
    yhi = jnp.minimum(jnp.floor(my - 0.5 + rcut), float(H - 1))
    empty = ylo > yhi
    b0 = (ylo.astype(jnp.int32) // HB)
    b1 = (yhi.astype(jnp.int32) // HB)
    b0_ref[...] = jnp.where(empty, NB + 1, b0)
    b1_ref[...] = jnp.where(empty, 0, b1)
    _ = mx_ref


def _bin_body(b0_hbm, b1_hbm, idx_hbm, cnt_hbm, b0_v, b1_v, idx_v,
              cnt_v, sem):
    c = jax.lax.axis_index("c")
    s = jax.lax.axis_index("s")
    u = s * 2 + c
    band_local = u // NSEG
    band = band_local
    seg = u % NSEG
    gbase = seg * SEG
    pltpu.sync_copy(b0_hbm.at[pl.ds(gbase, SEG)], b0_v)
    pltpu.sync_copy(b1_hbm.at[pl.ds(gbase, SEG)], b1_v)

    def chunk(i, ptr):
        b0c = b0_v[pl.ds(i * 16, 16)]
        b1c = b1_v[pl.ds(i * 16, 16)]
        mask = (b0c <= band) & (band <= b1c)
        mi = jnp.where(mask, 1, 0).astype(jnp.int32)
        pos = jax.lax.cumsum(mi, axis=0)
        offs = pos + (ptr - 1)
        gidx = jax.lax.iota(jnp.int32, 16) + (gbase + i * 16)
        plsc.store_scatter(idx_v, [offs], gidx, mask=mask)
        return ptr + jnp.sum(mi)

    ptr = jax.lax.fori_loop(0, SEG // 16, chunk, jnp.int32(0))
    lane = jax.lax.iota(jnp.int32, 16)
    plsc.store_scatter(idx_v, [ptr + lane], jnp.full((16,), N, jnp.int32),
                       mask=lane < SPAD)
    cnt_v[...] = jnp.full((16,), ptr, jnp.int32)
    pltpu.sync_copy(idx_v, idx_hbm.at[band_local, seg])
    pltpu.sync_copy(cnt_v, cnt_hbm.at[band_local, seg])
    _ = sem


@functools.lru_cache(maxsize=1)
def _make_bin_lists():
    return pl.kernel(
        _bin_body,
        out_type=(
            jax.ShapeDtypeStruct((NB, NSEG, CAP), jnp.int32),
            jax.ShapeDtypeStruct((NB, NSEG, 16), jnp.int32),
        ),
        mesh=plsc.VectorSubcoreMesh(core_axis_name="c", subcore_axis_name="s"),
        compiler_params=pltpu.CompilerParams(needs_layout_passes=False),
        scratch_types=[
            pltpu.VMEM((SEG,), jnp.int32),
            pltpu.VMEM((SEG,), jnp.int32),
            pltpu.VMEM((CAP,), jnp.int32),
            pltpu.VMEM((16,), jnp.int32),
            pltpu.SemaphoreType.DMA,
        ],
    )


def _bin_lists(b0, b1):
    return _make_bin_lists()(b0, b1)


def _raster_body(a2_ref, b2_ref, c2_ref, mx_ref, my_ref, op_ref,
                 cr_ref, cg_ref, cb_ref, idx_ref, cnt_ref, bg_ref, out_ref):
    b = pl.program_id(0)
    y0 = (b * HB).astype(jnp.float32) + 0.5
    py = jax.lax.broadcasted_iota(jnp.int32, (HB, W), 0).astype(jnp.float32) + y0
    px = jax.lax.broadcasted_iota(jnp.int32, (HB, W), 1).astype(jnp.float32) + 0.5

    def group(seg, jg, carry):
        T, ra, ga, ba = carry
        als = []
        cols = []
        base = jg * UNROLL
        for k in range(UNROLL):
            g = idx_ref[b, seg, base + k]
            dx = px - mx_ref[g]
            dy = py - my_ref[g]
            pw = dx * dx * a2_ref[g] + dy * dy * c2_ref[g] + dx * dy * b2_ref[g]
            al = op_ref[g] * jnp.exp(pw)
            al = jnp.where(al < _INV255, 0.0, al)
            als.append(al)
            cols.append((cr_ref[g], cg_ref[g], cb_ref[g]))
        q = [1.0 - al for al in als]
        cache = {}

        def rp(i, j):
            if (i, j) not in cache:
                if j - i == 1:
                    cache[(i, j)] = q[i]
                else:
                    m = i + (1 << ((j - i - 1).bit_length() - 1))
                    cache[(i, j)] = rp(i, m) * rp(m, j)
            return cache[(i, j)]

        us = [als[0]] + [als[k] * rp(0, k) for k in range(1, UNROLL)]

        def tree_sum(vals):
            vals = list(vals)
            while len(vals) > 1:
                nxt = [vals[i] + vals[i + 1] for i in range(0, len(vals) - 1, 2)]
                if len(vals) % 2:
                    nxt.append(vals[-1])
                vals = nxt
            return vals[0]

        sr = tree_sum(us[k] * cols[k][0] for k in range(UNROLL))
        sg = tree_sum(us[k] * cols[k][1] for k in range(UNROLL))
        sb = tree_sum(us[k] * cols[k][2] for k in range(UNROLL))
        ra = ra + T * sr
        ga = ga + T * sg
        ba = ba + T * sb
        T = T * rp(0, UNROLL)
        return (T, ra, ga, ba)

    ones = jnp.ones((HB, W), jnp.float32)
    zeros = jnp.zeros((HB, W), jnp.float32)
    carry = (ones, zeros, zeros, zeros)
    for seg in range(NSEG):
        n = cnt_ref[b, seg, 0]
        ngroups = (n + UNROLL - 1) // UNROLL
        carry = jax.lax.fori_loop(0, ngroups, functools.partial(group, seg),
                                  carry)
    T, ra, ga, ba = carry
    out_ref[0] = ra + T * bg_ref[0]
    out_ref[1] = ga + T * bg_ref[1]
    out_ref[2] = ba + T * bg_ref[2]


def kernel(means2D, opacities, colors, scale, rots, bg):
    f32 = jnp.float32
    mx2 = means2D[:, 0].reshape(PR, PC)
    my2 = means2D[:, 1].reshape(PR, PC)
    op2 = opacities[:, 0].reshape(PR, PC)
    sx2 = scale[:, 0].reshape(PR, PC)
    sy2 = scale[:, 1].reshape(PR, PC)
    th2 = rots[:, 0].reshape(PR, PC)

    a2, b2, c2, b0, b1, radii2 = pl.pallas_call(
        _prep_body,
        out_shape=(
            jax.ShapeDtypeStruct((PR, PC), f32),
            jax.ShapeDtypeStruct((PR, PC), f32),
            jax.ShapeDtypeStruct((PR, PC), f32),
            jax.ShapeDtypeStruct((PR, PC), jnp.int32),
            jax.ShapeDtypeStruct((PR, PC), jnp.int32),
            jax.ShapeDtypeStruct((PR, PC), jnp.int32),
        ),
    )(mx2, my2, op2, sx2, sy2, th2)

    b0r = b0.reshape(N)
    b1r = b1.reshape(N)

    pad = jnp.zeros((8,), f32)
    def _p(x):
        return jnp.concatenate([x.reshape(N), pad])
    params = (
        _p(a2), _p(b2), _p(c2),
        _p(mx2), _p(my2), _p(op2),
        _p(colors[:, 0]), _p(colors[:, 1]), _p(colors[:, 2]),
    )

    smem = pl.BlockSpec(memory_space=pltpu.SMEM)

    idx, cnt = _bin_lists(b0r, b1r)
    out = pl.pallas_call(
        _raster_body,
        grid=(NB,),
        in_specs=[smem] * 12,
        out_specs=pl.BlockSpec((3, HB, W), lambda b: (0, b, 0)),
        out_shape=jax.ShapeDtypeStruct((3, H, W), f32),
        compiler_params=pltpu.CompilerParams(
            dimension_semantics=("arbitrary",),
        ),
    )(*params, idx, cnt, bg)
    return (out, radii2.reshape(N))


# speedup vs baseline: 1.0716x; 1.0224x over previous
import functools

import jax
import jax.numpy as jnp
from jax.experimental import pallas as pl
from jax.experimental.pallas import tpu as pltpu
from jax.experimental.pallas import tpu_sc as plsc

H = 128
W = 128
N = 2048
HB = 16
NB = H // HB
NSEG = 4
SEG = N // NSEG
PR = 16
PC = N // PR
UNROLL = 16
SPAD = 16
CAP = SEG + SPAD

_INV255 = 1.0 / 255.0


def _prep_body(mx_ref, my_ref, op_ref, sx_ref, sy_ref, th_ref,
               a2_ref, b2_ref, c2_ref, b0_ref, b1_ref, radii_ref):
    th = th_ref[...]
    c = jnp.cos(th)
    s = jnp.sin(th)
    sx2 = sx_ref[...] ** 2
    sy2 = sy_ref[...] ** 2
    Sxx = c * c * sx2 + s * s * sy2 + 0.3
    Sxy = c * s * (sx2 - sy2)
    Syy = s * s * sx2 + c * c * sy2 + 0.3
    det = Sxx * Syy - Sxy * Sxy
    inv_det = 1.0 / det
    a2_ref[...] = -0.5 * Syy * inv_det
    b2_ref[...] = Sxy * inv_det
    c2_ref[...] = -0.5 * Sxx * inv_det
    mid = 0.5 * (Sxx + Syy)
    lam = mid + jnp.sqrt(jnp.maximum(mid * mid - det, 0.1))
    radii_ref[...] = jnp.ceil(3.0 * jnp.sqrt(lam)).astype(jnp.int32)
    op = op_ref[...]
    log_t = jnp.log(jnp.maximum(op, 1e-30) * 255.0)
    rcut = jnp.sqrt(2.0 * Syy * jnp.maximum(log_t, 0.0)) * 1.001 + 0.01
    my = my_ref[...]
    ylo = jnp.maximum(jnp.ceil(my - 0.5 - rcut), 0.0)
    yhi = jnp.minimum(jnp.floor(my - 0.5 + rcut), float(H - 1))
    empty = ylo > yhi
    b0 = (ylo.astype(jnp.int32) // HB)
    b1 = (yhi.astype(jnp.int32) // HB)
    b0_ref[...] = jnp.where(empty, NB + 1, b0)
    b1_ref[...] = jnp.where(empty, 0, b1)
    _ = mx_ref


def _bin_body(b0_hbm, b1_hbm, idx_hbm, cnt_hbm, b0_v, b1_v, idx_v,
              cnt_v, sem):
    c = jax.lax.axis_index("c")
    s = jax.lax.axis_index("s")
    u = s * 2 + c
    band_local = u // NSEG
    band = band_local
    seg = u % NSEG
    gbase = seg * SEG
    pltpu.sync_copy(b0_hbm.at[pl.ds(gbase, SEG)], b0_v)
    pltpu.sync_copy(b1_hbm.at[pl.ds(gbase, SEG)], b1_v)

    def chunk(i, ptr):
        b0c = b0_v[pl.ds(i * 16, 16)]
        b1c = b1_v[pl.ds(i * 16, 16)]
        mask = (b0c <= band) & (band <= b1c)
        mi = jnp.where(mask, 1, 0).astype(jnp.int32)
        pos = jax.lax.cumsum(mi, axis=0)
        offs = pos + (ptr - 1)
        gidx = jax.lax.iota(jnp.int32, 16) + (gbase + i * 16)
        plsc.store_scatter(idx_v, [offs], gidx, mask=mask)
        return ptr + jnp.sum(mi)

    ptr = jax.lax.fori_loop(0, SEG // 16, chunk, jnp.int32(0))
    lane = jax.lax.iota(jnp.int32, 16)
    plsc.store_scatter(idx_v, [ptr + lane], jnp.full((16,), N, jnp.int32),
                       mask=lane < SPAD)
    cnt_v[...] = jnp.full((16,), ptr, jnp.int32)
    pltpu.sync_copy(idx_v, idx_hbm.at[band_local, seg])
    pltpu.sync_copy(cnt_v, cnt_hbm.at[band_local, seg])
    _ = sem


@functools.lru_cache(maxsize=1)
def _make_bin_lists():
    return pl.kernel(
        _bin_body,
        out_type=(
            jax.ShapeDtypeStruct((NB, NSEG, CAP), jnp.int32),
            jax.ShapeDtypeStruct((NB, NSEG, 16), jnp.int32),
        ),
        mesh=plsc.VectorSubcoreMesh(core_axis_name="c", subcore_axis_name="s"),
        compiler_params=pltpu.CompilerParams(needs_layout_passes=False),
        scratch_types=[
            pltpu.VMEM((SEG,), jnp.int32),
            pltpu.VMEM((SEG,), jnp.int32),
            pltpu.VMEM((CAP,), jnp.int32),
            pltpu.VMEM((16,), jnp.int32),
            pltpu.SemaphoreType.DMA,
        ],
    )


def _bin_lists(b0, b1):
    return _make_bin_lists()(b0, b1)


def _raster_body(a2_ref, b2_ref, c2_ref, mx_ref, my_ref, op_ref,
                 cr_ref, cg_ref, cb_ref, idx_ref, cnt_ref, bg_ref, out_ref):
    b = pl.program_id(0)
    y0 = (b * HB).astype(jnp.float32) + 0.5
    py = jax.lax.broadcasted_iota(jnp.int32, (HB, W), 0).astype(jnp.float32) + y0
    px = jax.lax.broadcasted_iota(jnp.int32, (HB, W), 1).astype(jnp.float32) + 0.5

    def group(seg, jg, carry):
        T, ra, ga, ba = carry
        als = []
        cols = []
        base = jg * UNROLL
        for k in range(UNROLL):
            g = idx_ref[b, seg, base + k]
            dx = px - mx_ref[g]
            dy = py - my_ref[g]
            pw = dx * dx * a2_ref[g] + dy * dy * c2_ref[g] + dx * dy * b2_ref[g]
            al = op_ref[g] * jnp.exp(pw)
            al = jnp.where(al < _INV255, 0.0, al)
            als.append(al)
            cols.append((cr_ref[g], cg_ref[g], cb_ref[g]))
        q = [1.0 - al for al in als]
        cache = {}

        def rp(i, j):
            if (i, j) not in cache:
                if j - i == 1:
                    cache[(i, j)] = q[i]
                else:
                    m = i + (1 << ((j - i - 1).bit_length() - 1))
                    cache[(i, j)] = rp(i, m) * rp(m, j)
            return cache[(i, j)]

        us = [als[0]] + [als[k] * rp(0, k) for k in range(1, UNROLL)]

        def tree_sum(vals):
            vals = list(vals)
            while len(vals) > 1:
                nxt = [vals[i] + vals[i + 1] for i in range(0, len(vals) - 1, 2)]
                if len(vals) % 2:
                    nxt.append(vals[-1])
                vals = nxt
            return vals[0]

        sr = tree_sum(us[k] * cols[k][0] for k in range(UNROLL))
        sg = tree_sum(us[k] * cols[k][1] for k in range(UNROLL))
        sb = tree_sum(us[k] * cols[k][2] for k in range(UNROLL))
        ra = ra + T * sr
        ga = ga + T * sg
        ba = ba + T * sb
        T = T * rp(0, UNROLL)
        return (T, ra, ga, ba)

    ones = jnp.ones((HB, W), jnp.float32)
    zeros = jnp.zeros((HB, W), jnp.float32)
    carry = (ones, zeros, zeros, zeros)
    for seg in range(NSEG):
        n = cnt_ref[b, seg, 0]
        ngroups = (n + UNROLL - 1) // UNROLL
        carry = jax.lax.fori_loop(0, ngroups, functools.partial(group, seg),
                                  carry)
    T, ra, ga, ba = carry
    out_ref[0] = ra + T * bg_ref[0]
    out_ref[1] = ga + T * bg_ref[1]
    out_ref[2] = ba + T * bg_ref[2]


def kernel(means2D, opacities, colors, scale, rots, bg):
    f32 = jnp.float32
    mx2 = means2D[:, 0].reshape(PR, PC)
    my2 = means2D[:, 1].reshape(PR, PC)
    op2 = opacities[:, 0].reshape(PR, PC)
    sx2 = scale[:, 0].reshape(PR, PC)
    sy2 = scale[:, 1].reshape(PR, PC)
    th2 = rots[:, 0].reshape(PR, PC)

    a2, b2, c2, b0, b1, radii2 = pl.pallas_call(
        _prep_body,
        out_shape=(
            jax.ShapeDtypeStruct((PR, PC), f32),
            jax.ShapeDtypeStruct((PR, PC), f32),
            jax.ShapeDtypeStruct((PR, PC), f32),
            jax.ShapeDtypeStruct((PR, PC), jnp.int32),
            jax.ShapeDtypeStruct((PR, PC), jnp.int32),
            jax.ShapeDtypeStruct((PR, PC), jnp.int32),
        ),
    )(mx2, my2, op2, sx2, sy2, th2)

    b0r = b0.reshape(N)
    b1r = b1.reshape(N)

    pad = jnp.zeros((8,), f32)
    def _p(x):
        return jnp.concatenate([x.reshape(N), pad])
    params = (
        _p(a2), _p(b2), _p(c2),
        _p(mx2), _p(my2), _p(op2),
        _p(colors[:, 0]), _p(colors[:, 1]), _p(colors[:, 2]),
    )

    smem = pl.BlockSpec(memory_space=pltpu.SMEM)

    idx, cnt = _bin_lists(b0r, b1r)
    out = pl.pallas_call(
        _raster_body,
        grid=(NB,),
        in_specs=[smem] * 12,
        out_specs=pl.BlockSpec((3, HB, W), lambda b: (0, b, 0)),
        out_shape=jax.ShapeDtypeStruct((3, H, W), f32),
        compiler_params=pltpu.CompilerParams(
            dimension_semantics=("arbitrary",),
        ),
    )(*params, idx, cnt, bg)
    return (out, radii2.reshape(N))
